# Initial kernel scaffold; baseline (speedup 1.0000x reference)
import jax
import jax.numpy as jnp
from jax.experimental import pallas as pl


def kernel(img, node_embeddings, edge_index, edge_weight, W_g1, b_g1, W_g2, b_g2, W_m1, b_m1, W_m2, b_m2):
    raise NotImplementedError("write your pallas kernel here")


# speedup vs baseline: 1.0258x; 1.0258x over previous
import functools

import jax
import jax.numpy as jnp
from jax.experimental import pallas as pl
from jax.experimental.pallas import tpu as pltpu

_N_ATTRS = 115
_N_OBJS = 245
_N_PAIRS = 28175
_N_NODES = _N_ATTRS + _N_OBJS + _N_PAIRS
_SCALE = 50.0


def _dense_gcn_body(agg_ref, w1_ref, b1_ref, w2_ref, b2_ref, out_ref):
    h = jnp.dot(agg_ref[...], w1_ref[...], preferred_element_type=jnp.float32)
    h = jnp.maximum(h + b1_ref[...], 0.0)
    out_ref[...] = jnp.dot(h, w2_ref[...], preferred_element_type=jnp.float32) + b2_ref[...]


def _dense_gcn(agg, W1, b1, W2, b2):
    n, k = agg.shape
    hid = W1.shape[1]
    emb = W2.shape[1]
    blk = 1024
    return pl.pallas_call(
        _dense_gcn_body,
        grid=(pl.cdiv(n, blk),),
        in_specs=[
            pl.BlockSpec((blk, k), lambda i: (i, 0)),
            pl.BlockSpec((k, hid), lambda i: (0, 0)),
            pl.BlockSpec((1, hid), lambda i: (0, 0)),
            pl.BlockSpec((hid, emb), lambda i: (0, 0)),
            pl.BlockSpec((1, emb), lambda i: (0, 0)),
        ],
        out_specs=pl.BlockSpec((blk, emb), lambda i: (i, 0)),
        out_shape=jax.ShapeDtypeStruct((n, emb), jnp.float32),
    )(agg, W1, b1, W2, b2)


def _img_mlp_body(img_ref, w1_ref, b1_ref, w2_ref, b2_ref, out_ref):
    f = jnp.dot(img_ref[...], w1_ref[...], preferred_element_type=jnp.float32)
    f = jnp.maximum(f + b1_ref[...], 0.0)
    f = jnp.dot(f, w2_ref[...], preferred_element_type=jnp.float32) + b2_ref[...]
    nrm = jnp.sqrt(jnp.sum(f * f, axis=1, keepdims=True))
    out_ref[...] = f / (nrm + 1e-8)


def _img_mlp(img, W1, b1, W2, b2):
    b, feat = img.shape
    hid = W1.shape[1]
    emb = W2.shape[1]
    return pl.pallas_call(
        _img_mlp_body,
        out_shape=jax.ShapeDtypeStruct((b, emb), jnp.float32),
    )(img, W1, b1, W2, b2)


def _scores_body(fhat_ref, p_ref, out_ref):
    p = p_ref[...]
    nrm = jnp.sqrt(jnp.sum(p * p, axis=1, keepdims=True))
    ph = p / (nrm + 1e-8)
    out_ref[...] = _SCALE * jax.lax.dot_general(
        fhat_ref[...], ph, (((1,), (1,)), ((), ())),
        preferred_element_type=jnp.float32)


def _scores(fhat, pairs):
    npair, emb = pairs.shape
    b = fhat.shape[0]
    blk = 2048
    return pl.pallas_call(
        _scores_body,
        grid=(pl.cdiv(npair, blk),),
        in_specs=[
            pl.BlockSpec((b, emb), lambda i: (0, 0)),
            pl.BlockSpec((blk, emb), lambda i: (i, 0)),
        ],
        out_specs=pl.BlockSpec((b, blk), lambda i: (0, i)),
        out_shape=jax.ShapeDtypeStruct((b, npair), jnp.float32),
    )(fhat, pairs)


def kernel(img, node_embeddings, edge_index, edge_weight,
           W_g1, b_g1, W_g2, b_g2, W_m1, b_m1, W_m2, b_m2):
    src = edge_index[0]
    dst = edge_index[1]
    deg = jax.ops.segment_sum(edge_weight, dst, num_segments=_N_NODES)
    d_inv = jnp.where(deg > 0, jax.lax.rsqrt(deg), 0.0)
    w_norm = edge_weight * d_inv[src] * d_inv[dst]

    embp = jnp.pad(node_embeddings, ((0, 0), (0, 20)))
    agg = jax.ops.segment_sum(embp[src] * w_norm[:, None], dst,
                              num_segments=_N_NODES)
    W1p = jnp.pad(W_g1, ((0, 20), (0, 0)))
    hW = _dense_gcn(agg, W1p, b_g1.reshape(1, -1), W_g2, b_g2.reshape(1, -1))

    h2 = jax.ops.segment_sum(hW[src] * w_norm[:, None], dst,
                             num_segments=_N_NODES)
    pair_embeds = h2[_N_ATTRS + _N_OBJS:]

    fhat = _img_mlp(img, W_m1, b_m1.reshape(1, -1), W_m2, b_m2.reshape(1, -1))
    return _scores(fhat, pair_embeds)


# speedup vs baseline: 1.5099x; 1.4719x over previous
import functools

import jax
import jax.numpy as jnp
from jax import lax
from jax.experimental import pallas as pl
from jax.experimental.pallas import tpu as pltpu
from jax.experimental.pallas import tpu_sc as plsc

_N_ATTRS = 115
_N_OBJS = 245
_N_PAIRS = 28175
_N_NODES = _N_ATTRS + _N_OBJS + _N_PAIRS
_SCALE = 50.0

_E = 197585
_E_PAD = 198656
_SCAN = 1552
_PER_TILE = 12416
_N16 = 28544
_RA = 14336
_NDEG = 2 * _RA
_DUMP = 2048

_SC_MESH = plsc.VectorSubcoreMesh(
    core_axis_name="c", subcore_axis_name="s", num_cores=2, num_subcores=16)
_SC_PARAMS = pltpu.CompilerParams(needs_layout_passes=False)


def _iota16():
    return lax.iota(jnp.int32, 16)


def _deg_body(dst_hbm, w_hbm, deg_out, scan_dst, scan_w, cdst, cw,
              vrows, dbuf, obuf, acc):
    c = lax.axis_index("c")
    s = lax.axis_index("s")
    lo = c * _RA
    zeros16 = jnp.zeros((16,), jnp.int32)

    def zv(i, _):
        vrows[i, :] = jnp.zeros((16,), jnp.float32)
        return 0
    lax.fori_loop(0, 128, zv, 0)
    for m in range(7):
        pltpu.sync_copy(vrows, acc.at[pl.ds(s * 896 + m * 128, 128), :])

    def pf(q, _):
        cdst[q >> 3, pl.ds(16 * (q & 7), 16)] = _RA + s * 128 + ((_iota16() + 16 * q) & 127)
        return 0
    lax.fori_loop(0, 97 * 8, pf, 0)

    def scan_chunk(sc, wp):
        base = s * _PER_TILE + sc * _SCAN
        pltpu.sync_copy(dst_hbm.at[pl.ds(base, _SCAN)], scan_dst)
        pltpu.sync_copy(w_hbm.at[pl.ds(base, _SCAN)], scan_w)

        def step(k, wp):
            d16 = scan_dst[pl.ds(16 * k, 16)]
            m = (d16 >= lo) & (d16 < lo + _RA)
            mi = m.astype(jnp.int32)
            pos = jnp.maximum(wp + plsc.cumsum(mi) - 1, 0)
            plsc.store_scatter(cdst, [pos >> 7, pos & 127], d16 - lo, mask=m)
            plsc.store_scatter(cw, [pos], scan_w[pl.ds(16 * k, 16)], mask=m)
            return wp + jnp.sum(mi)
        return lax.fori_loop(0, _SCAN // 16, step, wp)
    count = lax.fori_loop(0, _PER_TILE // _SCAN, scan_chunk, jnp.int32(0))

    plsc.subcore_barrier()

    for seg in range(97):
        @pl.when(seg * 128 < count)
        def _():
            sbase = seg * 128

            def build(j, _):
                w16 = cw[pl.ds(sbase + 16 * j, 16)]
                plsc.store_scatter(vrows, [_iota16() + 16 * j, zeros16], w16)
                return 0
            lax.fori_loop(0, 8, build, 0)
            pltpu.sync_copy(vrows, acc.at[cdst.at[seg]], add=True)

    plsc.subcore_barrier()

    for m in range(7):
        pltpu.sync_copy(acc.at[pl.ds(s * 896 + m * 128, 128), :], dbuf)

        def ext(j, _):
            obuf[pl.ds(16 * j, 16)] = plsc.load_gather(
                dbuf, [_iota16() + 16 * j, zeros16])
            return 0
        lax.fori_loop(0, 8, ext, 0)
        pltpu.sync_copy(obuf, deg_out.at[pl.ds(c * _RA + s * 896 + m * 128, 128)])


def _deg_kernel(dst_pad, w_pad):
    return pl.kernel(
        _deg_body,
        out_type=jax.ShapeDtypeStruct((_NDEG,), jnp.float32),
        mesh=_SC_MESH,
        compiler_params=_SC_PARAMS,
        scratch_types=[
            pltpu.VMEM((_SCAN,), jnp.int32),
            pltpu.VMEM((_SCAN,), jnp.float32),
            pltpu.VMEM((97, 128), jnp.int32),
            pltpu.VMEM((_PER_TILE + 16,), jnp.float32),
            pltpu.VMEM((128, 16), jnp.float32),
            pltpu.VMEM((128, 16), jnp.float32),
            pltpu.VMEM((128,), jnp.float32),
            pltpu.VMEM_SHARED((_RA + _DUMP, 16), jnp.float32),
        ],
    )(dst_pad, w_pad)


def _wnorm_body(deg_hbm, src_hbm, dst_hbm, w_hbm, wn_out,
                degv, dinv, scan_src, scan_dst, scan_w, wout):
    c = lax.axis_index("c")
    s = lax.axis_index("s")
    wid = s * 2 + c

    pltpu.sync_copy(deg_hbm.at[pl.ds(0, _NDEG)], degv)

    half = jnp.full((16,), 0.5, jnp.float32)
    three_half = jnp.full((16,), 1.5, jnp.float32)
    magic = jnp.full((16,), 0x5F3759DF, jnp.int32)

    def newton(i, _):
        x = degv[pl.ds(16 * i, 16)]
        bits = lax.bitcast_convert_type(x, jnp.int32)
        y = lax.bitcast_convert_type(
            magic - lax.shift_right_logical(bits, 1), jnp.float32)
        for _unused in range(3):
            y = y * (three_half - half * x * y * y)
        dinv[pl.ds(16 * i, 16)] = jnp.where(x > 0.0, y, 0.0)
        return 0
    lax.fori_loop(0, _NDEG // 16, newton, 0)

    per_tile = _E_PAD // 32

    def chunk(sc, _):
        base = wid * per_tile + sc * _SCAN
        pltpu.sync_copy(src_hbm.at[pl.ds(base, _SCAN)], scan_src)
        pltpu.sync_copy(dst_hbm.at[pl.ds(base, _SCAN)], scan_dst)
        pltpu.sync_copy(w_hbm.at[pl.ds(base, _SCAN)], scan_w)

        def step(k, _):
            s16 = scan_src[pl.ds(16 * k, 16)]
            d16 = scan_dst[pl.ds(16 * k, 16)]
            a = plsc.load_gather(dinv, [s16])
            b = plsc.load_gather(dinv, [d16])
            wout[pl.ds(16 * k, 16)] = scan_w[pl.ds(16 * k, 16)] * a * b
            return 0
        lax.fori_loop(0, _SCAN // 16, step, 0)
        pltpu.sync_copy(wout, wn_out.at[pl.ds(base, _SCAN)])
        return 0
    lax.fori_loop(0, per_tile // _SCAN, chunk, 0)


def _wnorm_kernel(deg, src_pad, dst_pad, w_pad):
    return pl.kernel(
        _wnorm_body,
        out_type=jax.ShapeDtypeStruct((_E_PAD,), jnp.float32),
        mesh=_SC_MESH,
        compiler_params=_SC_PARAMS,
        scratch_types=[
            pltpu.VMEM((_NDEG,), jnp.float32),
            pltpu.VMEM((_NDEG,), jnp.float32),
            pltpu.VMEM((_SCAN,), jnp.int32),
            pltpu.VMEM((_SCAN,), jnp.int32),
            pltpu.VMEM((_SCAN,), jnp.float32),
            pltpu.VMEM((_SCAN,), jnp.float32),
        ],
    )(deg, src_pad, dst_pad, w_pad)


def _dense_gcn_body(agg_ref, w1_ref, b1_ref, w2_ref, b2_ref, out_ref):
    h = jnp.dot(agg_ref[...], w1_ref[...], preferred_element_type=jnp.float32)
    h = jnp.maximum(h + b1_ref[...], 0.0)
    out_ref[...] = jnp.dot(h, w2_ref[...], preferred_element_type=jnp.float32) + b2_ref[...]


def _dense_gcn(agg, W1, b1, W2, b2):
    n, k = agg.shape
    hid = W1.shape[1]
    emb = W2.shape[1]
    blk = 1024
    return pl.pallas_call(
        _dense_gcn_body,
        grid=(pl.cdiv(n, blk),),
        in_specs=[
            pl.BlockSpec((blk, k), lambda i: (i, 0)),
            pl.BlockSpec((k, hid), lambda i: (0, 0)),
            pl.BlockSpec((1, hid), lambda i: (0, 0)),
            pl.BlockSpec((hid, emb), lambda i: (0, 0)),
            pl.BlockSpec((1, emb), lambda i: (0, 0)),
        ],
        out_specs=pl.BlockSpec((blk, emb), lambda i: (i, 0)),
        out_shape=jax.ShapeDtypeStruct((n, emb), jnp.float32),
    )(agg, W1, b1, W2, b2)


def _img_mlp_body(img_ref, w1_ref, b1_ref, w2_ref, b2_ref, out_ref):
    f = jnp.dot(img_ref[...], w1_ref[...], preferred_element_type=jnp.float32)
    f = jnp.maximum(f + b1_ref[...], 0.0)
    f = jnp.dot(f, w2_ref[...], preferred_element_type=jnp.float32) + b2_ref[...]
    nrm = jnp.sqrt(jnp.sum(f * f, axis=1, keepdims=True))
    out_ref[...] = f / (nrm + 1e-8)


def _img_mlp(img, W1, b1, W2, b2):
    b, feat = img.shape
    hid = W1.shape[1]
    emb = W2.shape[1]
    return pl.pallas_call(
        _img_mlp_body,
        out_shape=jax.ShapeDtypeStruct((b, emb), jnp.float32),
    )(img, W1, b1, W2, b2)


def _scores_body(fhat_ref, p_ref, out_ref):
    p = p_ref[...]
    nrm = jnp.sqrt(jnp.sum(p * p, axis=1, keepdims=True))
    ph = p / (nrm + 1e-8)
    out_ref[...] = _SCALE * jax.lax.dot_general(
        fhat_ref[...], ph, (((1,), (1,)), ((), ())),
        preferred_element_type=jnp.float32)


def _scores(fhat, pairs):
    npair, emb = pairs.shape
    b = fhat.shape[0]
    blk = 2048
    return pl.pallas_call(
        _scores_body,
        grid=(pl.cdiv(npair, blk),),
        in_specs=[
            pl.BlockSpec((b, emb), lambda i: (0, 0)),
            pl.BlockSpec((blk, emb), lambda i: (i, 0)),
        ],
        out_specs=pl.BlockSpec((b, blk), lambda i: (0, i)),
        out_shape=jax.ShapeDtypeStruct((b, npair), jnp.float32),
    )(fhat, pairs)


def kernel(img, node_embeddings, edge_index, edge_weight,
           W_g1, b_g1, W_g2, b_g2, W_m1, b_m1, W_m2, b_m2):
    src = edge_index[0]
    dst = edge_index[1]
    pad_e = _E_PAD - _E
    src_pad = jnp.pad(src, (0, pad_e))
    dst_pad = jnp.pad(dst, (0, pad_e))
    w_pad = jnp.pad(edge_weight, (0, pad_e))
    deg = jax.ops.segment_sum(edge_weight, dst, num_segments=_N_NODES)
    deg_w = jnp.pad(deg, (0, _NDEG - _N_NODES))
    w_norm = _wnorm_kernel(deg_w, src_pad, dst_pad, w_pad)[:_E]

    embp = jnp.pad(node_embeddings, ((0, 0), (0, 20)))
    agg = jax.ops.segment_sum(embp[src] * w_norm[:, None], dst,
                              num_segments=_N_NODES)
    W1p = jnp.pad(W_g1, ((0, 20), (0, 0)))
    hW = _dense_gcn(agg, W1p, b_g1.reshape(1, -1), W_g2, b_g2.reshape(1, -1))

    h2 = jax.ops.segment_sum(hW[src] * w_norm[:, None], dst,
                             num_segments=_N_NODES)
    pair_embeds = h2[_N_ATTRS + _N_OBJS:]

    fhat = _img_mlp(img, W_m1, b_m1.reshape(1, -1), W_m2, b_m2.reshape(1, -1))
    return _scores(fhat, pair_embeds)
